# Initial kernel scaffold; baseline (speedup 1.0000x reference)
import jax
import jax.numpy as jnp
from jax.experimental import pallas as pl


def kernel(x, gate_weight, W_gate, W_up, W_down):
    raise NotImplementedError("write your pallas kernel here")


# speedup vs baseline: 2.5254x; 2.5254x over previous
import functools

import jax
import jax.numpy as jnp
from jax.experimental import pallas as pl
from jax.experimental.pallas import tpu as pltpu

H = 1024
FF = 2048
E = 8
TOPK = 2
FFE = FF // E


def _moe_tile_kernel(x_ref, gw_ref, wg_ref, wu_ref, wd_ref, o_ref):
    x = x_ref[...]
    logits = jnp.dot(x, gw_ref[...], preferred_element_type=jnp.float32)
    probs = jax.nn.softmax(logits, axis=-1)

    e_idx = jax.lax.broadcasted_iota(jnp.int32, probs.shape, 1)
    v1 = jnp.max(probs, axis=-1, keepdims=True)
    i1 = jnp.min(jnp.where(probs == v1, e_idx, E), axis=-1, keepdims=True)
    masked = jnp.where(e_idx == i1, -jnp.inf, probs)
    v2 = jnp.max(masked, axis=-1, keepdims=True)
    i2 = jnp.min(jnp.where(masked == v2, e_idx, E), axis=-1, keepdims=True)

    denom = jnp.clip(v1 + v2, 1e-9, None)
    w_full = (
        jnp.where(e_idx == i1, v1, 0.0) + jnp.where(e_idx == i2, v2, 0.0)
    ) / denom

    hg = jnp.dot(x, wg_ref[...], preferred_element_type=jnp.float32)
    hu = jnp.dot(x, wu_ref[...], preferred_element_type=jnp.float32)
    scale = jnp.broadcast_to(w_full[:, :, None], (x.shape[0], E, FFE)).reshape(
        x.shape[0], FF
    )
    h1 = (hg * jax.nn.sigmoid(hg)) * hu * scale
    o_ref[...] = jnp.dot(h1, wd_ref[...], preferred_element_type=jnp.float32)


@jax.jit
def kernel(x, gate_weight, W_gate, W_up, W_down):
    b, s, h = x.shape
    T = b * s
    x_flat = x.reshape(T, h)
    wg_all = W_gate.transpose(1, 0, 2).reshape(H, FF)
    wu_all = W_up.transpose(1, 0, 2).reshape(H, FF)
    wd_all = W_down.reshape(FF, H)

    TT = 256
    grid = (T // TT,)
    out = pl.pallas_call(
        _moe_tile_kernel,
        grid=grid,
        in_specs=[
            pl.BlockSpec((TT, H), lambda i: (i, 0)),
            pl.BlockSpec((H, E), lambda i: (0, 0)),
            pl.BlockSpec((H, FF), lambda i: (0, 0)),
            pl.BlockSpec((H, FF), lambda i: (0, 0)),
            pl.BlockSpec((FF, H), lambda i: (0, 0)),
        ],
        out_specs=pl.BlockSpec((TT, H), lambda i: (i, 0)),
        out_shape=jax.ShapeDtypeStruct((T, H), jnp.float32),
    )(x_flat, gate_weight, wg_all, wu_all, wd_all)
    return out.reshape(b, s, h)


# speedup vs baseline: 2.9650x; 1.1741x over previous
import functools

import jax
import jax.numpy as jnp
from jax.experimental import pallas as pl
from jax.experimental.pallas import tpu as pltpu

H = 1024
FF = 2048
E = 8
TOPK = 2
FFE = FF // E


def _moe_tile_kernel(x_ref, xb_ref, gw_ref, wg_ref, wu_ref, wd_ref, o_ref):
    x = x_ref[...]
    logits = jnp.dot(x, gw_ref[...], preferred_element_type=jnp.float32)
    probs = jax.nn.softmax(logits, axis=-1)

    e_idx = jax.lax.broadcasted_iota(jnp.int32, probs.shape, 1)
    v1 = jnp.max(probs, axis=-1, keepdims=True)
    i1 = jnp.min(jnp.where(probs == v1, e_idx, E), axis=-1, keepdims=True)
    masked = jnp.where(e_idx == i1, -jnp.inf, probs)
    v2 = jnp.max(masked, axis=-1, keepdims=True)
    i2 = jnp.min(jnp.where(masked == v2, e_idx, E), axis=-1, keepdims=True)

    denom = jnp.clip(v1 + v2, 1e-9, None)
    w_full = (
        jnp.where(e_idx == i1, v1, 0.0) + jnp.where(e_idx == i2, v2, 0.0)
    ) / denom

    xb = xb_ref[...]
    hg = jnp.dot(xb, wg_ref[...], preferred_element_type=jnp.float32)
    hu = jnp.dot(xb, wu_ref[...], preferred_element_type=jnp.float32)
    scale = jnp.broadcast_to(w_full[:, :, None], (x.shape[0], E, FFE)).reshape(
        x.shape[0], FF
    )
    h1 = (hg * jax.nn.sigmoid(hg)) * hu * scale
    o_ref[...] = jnp.dot(
        h1.astype(jnp.bfloat16), wd_ref[...], preferred_element_type=jnp.float32
    )


@jax.jit
def kernel(x, gate_weight, W_gate, W_up, W_down):
    b, s, h = x.shape
    T = b * s
    x_flat = x.reshape(T, h)
    xb_flat = x_flat.astype(jnp.bfloat16)
    wg_all = W_gate.transpose(1, 0, 2).reshape(H, FF).astype(jnp.bfloat16)
    wu_all = W_up.transpose(1, 0, 2).reshape(H, FF).astype(jnp.bfloat16)
    wd_all = W_down.reshape(FF, H).astype(jnp.bfloat16)

    TT = 256
    grid = (T // TT,)
    out = pl.pallas_call(
        _moe_tile_kernel,
        grid=grid,
        in_specs=[
            pl.BlockSpec((TT, H), lambda i: (i, 0)),
            pl.BlockSpec((TT, H), lambda i: (i, 0)),
            pl.BlockSpec((H, E), lambda i: (0, 0)),
            pl.BlockSpec((H, FF), lambda i: (0, 0)),
            pl.BlockSpec((H, FF), lambda i: (0, 0)),
            pl.BlockSpec((FF, H), lambda i: (0, 0)),
        ],
        out_specs=pl.BlockSpec((TT, H), lambda i: (i, 0)),
        out_shape=jax.ShapeDtypeStruct((T, H), jnp.float32),
    )(x_flat, xb_flat, gate_weight, wg_all, wu_all, wd_all)
    return out.reshape(b, s, h)
